# baseline (speedup 1.0000x reference)
import jax
import jax.numpy as jnp
import numpy as np
from jax.experimental import pallas as pl
from jax.experimental.pallas import tpu as pltpu

B = 128
V = 100000
N = V - 1

VB = 12800
NVB = (V + VB - 1) // VB
CH = 64
UNROLL = 8
NITER = VB // (UNROLL * CH)
NITER_LAST = -(-(V - (NVB - 1) * VB) // (UNROLL * CH))

NGB = (N + VB - 1) // VB

_SEED = 42
_KS0 = 0
_KS1 = _SEED
_KS2 = _KS0 ^ _KS1 ^ 0x1BD11BDA
_KS = (_KS0, _KS1, _KS2)
_ROT = ((13, 15, 26, 6), (17, 29, 16, 24))
_TINY = np.float32(np.finfo(np.float32).tiny)
_NEG_INF = np.float32(-np.inf)


def _score(xx, x1):
    u32 = lambda c: jnp.uint32(c & 0xFFFFFFFF)
    x0 = x1
    for i in range(5):
        for j, r in enumerate(_ROT[i % 2]):
            if not (i == 0 and j == 0):
                x0 = x0 + x1
            x1 = (x1 << jnp.uint32(r)) | (x1 >> jnp.uint32(32 - r))
            x1 = x1 ^ x0
        if _KS[(i + 1) % 3]:
            x0 = x0 + u32(_KS[(i + 1) % 3])
        x1 = x1 + u32(_KS[(i + 2) % 3] + (i + 1))
    bits = x0 ^ x1
    f = jax.lax.bitcast_convert_type(
        (bits >> jnp.uint32(9)) | jnp.uint32(0x3F800000), jnp.float32
    ) - jnp.float32(1.0)
    return -jnp.log(-jnp.log(f)) + jnp.log(xx + jnp.float32(1e-20))


def _scan_kernel(xt_ref, c_ref, xo_ref, m_ref, idx_ref):
    step = pl.program_id(0)

    @pl.when(step == 0)
    def _init():
        m_ref[...] = jnp.full((CH, B), _NEG_INF, jnp.float32)
        idx_ref[...] = jnp.zeros((CH, B), jnp.int32)

    sub = jax.lax.broadcasted_iota(jnp.int32, (CH, B), 0)
    lane = jax.lax.broadcasted_iota(jnp.int32, (CH, B), 1)
    base42 = lane * V + sub + (step * VB + 42)
    v0 = step * VB

    def make_body(masked):
        def body(j, carry):
            m, idx = carry
            s0 = j * (UNROLL * CH)
            for k in range(UNROLL):
                sk = s0 + k * CH
                r = _score(xt_ref[pl.ds(sk, CH), :],
                           (base42 + sk).astype(jnp.uint32))
                if masked:
                    r = jnp.where(v0 + sk + sub < V, r, _NEG_INF)
                upd = r > m
                m = jnp.where(upd, r, m)
                idx = jnp.where(upd, step * (UNROLL * NITER) + UNROLL * j + k,
                                idx)
            return m, idx
        return body

    m, idx = jax.lax.cond(
        step == NVB - 1,
        lambda mi: jax.lax.fori_loop(0, NITER_LAST, make_body(True), mi),
        lambda mi: jax.lax.fori_loop(0, NITER, make_body(False), mi),
        (m_ref[...], idx_ref[...]),
    )
    m_ref[...] = m
    idx_ref[...] = idx

    xo_ref[...] = xt_ref[...]

    @pl.when(step == NVB - 1)
    def _finish():
        mm = m_ref[...]
        col = idx_ref[...] * CH + sub
        mrow = jnp.max(mm, axis=0, keepdims=True)
        c = jnp.min(jnp.where(mm == mrow, col, jnp.int32(V)),
                    axis=0, keepdims=True)
        c_ref[...] = jnp.broadcast_to(c, (8, B))


def _gate_kernel(c_ref, g_ref):
    step = pl.program_id(0)
    sub = jax.lax.broadcasted_iota(jnp.int32, (VB, B), 0)
    g_ref[...] = (step * VB + sub < c_ref[0:1, :]).astype(jnp.float32)


@jax.jit
def kernel(x):
    xt = x.T
    c8, xot = pl.pallas_call(
        _scan_kernel,
        grid=(NVB,),
        in_specs=[pl.BlockSpec((VB, B), lambda i: (i, 0))],
        out_specs=[
            pl.BlockSpec((8, B), lambda i: (0, 0)),
            pl.BlockSpec((VB, B), lambda i: (i, 0)),
        ],
        out_shape=[
            jax.ShapeDtypeStruct((8, B), jnp.int32),
            jax.ShapeDtypeStruct((V, B), jnp.float32),
        ],
        scratch_shapes=[
            pltpu.VMEM((CH, B), jnp.float32),
            pltpu.VMEM((CH, B), jnp.int32),
        ],
    )(xt)
    gt = pl.pallas_call(
        _gate_kernel,
        grid=(NGB,),
        in_specs=[pl.BlockSpec((8, B), lambda i: (0, 0))],
        out_specs=pl.BlockSpec((VB, B), lambda i: (i, 0)),
        out_shape=jax.ShapeDtypeStruct((N, B), jnp.float32),
    )(c8)
    return (gt.T, xot.T)


# speedup vs baseline: 1.0114x; 1.0114x over previous
import jax
import jax.numpy as jnp
import numpy as np
from jax.experimental import pallas as pl
from jax.experimental.pallas import tpu as pltpu

B = 128
V = 100000
N = V - 1

VB = 8192
NVB = (V + VB - 1) // VB
CH = 64
UNROLL = 16
NITER = VB // (UNROLL * CH)
NITER_LAST = -(-(V - (NVB - 1) * VB) // (UNROLL * CH))

NGB = (N + VB - 1) // VB

_SEED = 42
_KS0 = 0
_KS1 = _SEED
_KS2 = _KS0 ^ _KS1 ^ 0x1BD11BDA
_KS = (_KS0, _KS1, _KS2)
_ROT = ((13, 15, 26, 6), (17, 29, 16, 24))
_TINY = np.float32(np.finfo(np.float32).tiny)
_NEG_INF = np.float32(-np.inf)


def _score(xx, x1):
    u32 = lambda c: jnp.uint32(c & 0xFFFFFFFF)
    x0 = x1
    for i in range(5):
        for j, r in enumerate(_ROT[i % 2]):
            if not (i == 0 and j == 0):
                x0 = x0 + x1
            x1 = (x1 << jnp.uint32(r)) | (x1 >> jnp.uint32(32 - r))
            x1 = x1 ^ x0
        if _KS[(i + 1) % 3]:
            x0 = x0 + u32(_KS[(i + 1) % 3])
        x1 = x1 + u32(_KS[(i + 2) % 3] + (i + 1))
    bits = x0 ^ x1
    f = jax.lax.bitcast_convert_type(
        (bits >> jnp.uint32(9)) | jnp.uint32(0x3F800000), jnp.float32
    ) - jnp.float32(1.0)
    return -jnp.log(-jnp.log(f)) + jnp.log(xx + jnp.float32(1e-20))


def _scan_kernel(xt_ref, c_ref, xo_ref, m_ref, idx_ref):
    step = pl.program_id(0)

    @pl.when(step == 0)
    def _init():
        m_ref[...] = jnp.full((CH, B), _NEG_INF, jnp.float32)
        idx_ref[...] = jnp.zeros((CH, B), jnp.int32)

    sub = jax.lax.broadcasted_iota(jnp.int32, (CH, B), 0)
    lane = jax.lax.broadcasted_iota(jnp.int32, (CH, B), 1)
    base42 = lane * V + sub + (step * VB + 42)
    v0 = step * VB

    def make_body(masked):
        def body(j, carry):
            m, idx = carry
            s0 = j * (UNROLL * CH)
            for k in range(UNROLL):
                sk = s0 + k * CH
                r = _score(xt_ref[pl.ds(sk, CH), :],
                           (base42 + sk).astype(jnp.uint32))
                if masked:
                    r = jnp.where(v0 + sk + sub < V, r, _NEG_INF)
                upd = r > m
                m = jnp.where(upd, r, m)
                idx = jnp.where(upd, step * (UNROLL * NITER) + UNROLL * j + k,
                                idx)
            return m, idx
        return body

    m, idx = jax.lax.cond(
        step == NVB - 1,
        lambda mi: jax.lax.fori_loop(0, NITER_LAST, make_body(True), mi),
        lambda mi: jax.lax.fori_loop(0, NITER, make_body(False), mi),
        (m_ref[...], idx_ref[...]),
    )
    m_ref[...] = m
    idx_ref[...] = idx

    xo_ref[...] = xt_ref[...]

    @pl.when(step == NVB - 1)
    def _finish():
        mm = m_ref[...]
        col = idx_ref[...] * CH + sub
        mrow = jnp.max(mm, axis=0, keepdims=True)
        c = jnp.min(jnp.where(mm == mrow, col, jnp.int32(V)),
                    axis=0, keepdims=True)
        c_ref[...] = jnp.broadcast_to(c, (8, B))


def _gate_kernel(c_ref, g_ref):
    step = pl.program_id(0)
    sub = jax.lax.broadcasted_iota(jnp.int32, (VB, B), 0)
    g_ref[...] = (step * VB + sub < c_ref[0:1, :]).astype(jnp.float32)


@jax.jit
def kernel(x):
    xt = x.T
    c8, xot = pl.pallas_call(
        _scan_kernel,
        grid=(NVB,),
        in_specs=[pl.BlockSpec((VB, B), lambda i: (i, 0))],
        out_specs=[
            pl.BlockSpec((8, B), lambda i: (0, 0)),
            pl.BlockSpec((VB, B), lambda i: (i, 0)),
        ],
        out_shape=[
            jax.ShapeDtypeStruct((8, B), jnp.int32),
            jax.ShapeDtypeStruct((V, B), jnp.float32),
        ],
        scratch_shapes=[
            pltpu.VMEM((CH, B), jnp.float32),
            pltpu.VMEM((CH, B), jnp.int32),
        ],
    )(xt)
    gt = pl.pallas_call(
        _gate_kernel,
        grid=(NGB,),
        in_specs=[pl.BlockSpec((8, B), lambda i: (0, 0))],
        out_specs=pl.BlockSpec((VB, B), lambda i: (i, 0)),
        out_shape=jax.ShapeDtypeStruct((N, B), jnp.float32),
    )(c8)
    return (gt.T, xot.T)
